# baseline (speedup 1.0000x reference)
import functools

import jax
import jax.numpy as jnp
from jax import lax
from jax.experimental import pallas as pl
from jax.experimental.pallas import tpu as pltpu
from jax.experimental.pallas import tpu_sc as plsc

_B = 262144
_D = 256
_K = 5
_NC = 2
_NS = 16
_NW = _NC * _NS
_RW = _B // _NW
_C = 64
_G = _C // 16
_NCH = _RW // _C
_L = 16
_U = 8


def _round_bf16(v):
    u = plsc.bitcast(v, jnp.int32)
    r = (u + jnp.int32(0x8000)) & jnp.int32(-65536)
    return plsc.bitcast(r, jnp.float32)


def _sc_body(x_hbm, w_hbm, out_hbm, xbuf, wbuf, wrbuf, obuf, sem0, sem1):
    wid = lax.axis_index("c") * _NS + lax.axis_index("s")
    wbase = wid * _RW
    sems = (sem0, sem1)

    pltpu.sync_copy(w_hbm, wbuf)

    lanes = lax.iota(jnp.int32, _L)
    rowidx = [lanes + g * _L for g in range(_G)]
    ksplat = [jnp.broadcast_to(jnp.int32(k), (_L,)) for k in range(_K)]

    def wbody(d, w2accs):
        dsp = jnp.broadcast_to(d, (_L,))
        out = []
        for k in range(_K):
            wv = plsc.load_gather(wbuf, [ksplat[k], dsp])
            out.append(w2accs[k] + wv * wv)
        return tuple(out)

    w2s = lax.fori_loop(
        0, _D, wbody,
        tuple(jnp.zeros((_L,), jnp.float32) for _ in range(_K)))

    for k in range(_K):
        for v in range(_D // _L):
            wrbuf[k, pl.ds(v * _L, _L)] = _round_bf16(wbuf[k, pl.ds(v * _L, _L)])

    def fire(ci, slot):
        pltpu.make_async_copy(
            x_hbm.at[pl.ds(wbase + ci * _C, _C)],
            xbuf.at[slot],
            sems[slot],
        ).start()

    fire(0, 0)
    fire(1, 1)

    def process(ci, slot):
        pltpu.make_async_copy(
            x_hbm.at[pl.ds(wbase + ci * _C, _C)],
            xbuf.at[slot],
            sems[slot],
        ).wait()
        xr = xbuf.at[slot]

        def dbody(i, accs):
            for u in range(_U):
                d = i * _U + u
                dsp = jnp.broadcast_to(d, (_L,))
                wvals = [plsc.load_gather(wrbuf, [ksplat[k], dsp])
                         for k in range(_K)]
                out = []
                for g in range(_G):
                    xg = _round_bf16(plsc.load_gather(xr, [rowidx[g], dsp]))
                    out.append(tuple(accs[g][k] + xg * wvals[k]
                                     for k in range(_K)))
                accs = tuple(out)
            return accs

        init = tuple(
            tuple(jnp.zeros((_L,), jnp.float32) for _ in range(_K))
            for _ in range(_G)
        )
        accs = lax.fori_loop(0, _D // _U, dbody, init)

        for g in range(_G):
            best = w2s[0] - 2.0 * accs[g][0]
            bi = jnp.zeros((_L,), jnp.int32)
            for k in range(1, _K):
                sk = w2s[k] - 2.0 * accs[g][k]
                m = sk < best
                best = jnp.where(m, sk, best)
                bi = jnp.where(m, jnp.int32(k), bi)
            obuf[pl.ds(ci * _C + g * _L, _L)] = bi

        @pl.when(ci + 2 < _NCH)
        def _():
            fire(ci + 2, slot)

    def outer(i, carry):
        for b in range(2):
            process(2 * i + b, b)
        return carry

    lax.fori_loop(0, _NCH // 2, outer, 0)

    pltpu.sync_copy(obuf, out_hbm.at[pl.ds(wbase, _RW)])


@jax.jit
def kernel(x, weights):
    xf = x
    wf = weights
    f = pl.kernel(
        _sc_body,
        out_type=jax.ShapeDtypeStruct((_B,), jnp.int32),
        mesh=plsc.VectorSubcoreMesh(core_axis_name="c", subcore_axis_name="s"),
        compiler_params=pltpu.CompilerParams(
            needs_layout_passes=False,
            use_tc_tiling_on_sc=False,
        ),
        scratch_types=[
            pltpu.VMEM((2, _C, _D), jnp.float32),
            pltpu.VMEM((_K, _D), jnp.float32),
            pltpu.VMEM((_K, _D), jnp.float32),
            pltpu.VMEM((_RW,), jnp.int32),
            pltpu.SemaphoreType.DMA,
            pltpu.SemaphoreType.DMA,
        ],
    )
    return f(xf, wf)


# speedup vs baseline: 3.4894x; 3.4894x over previous
import functools

import jax
import jax.numpy as jnp
from jax import lax
from jax.experimental import pallas as pl
from jax.experimental.pallas import tpu as pltpu
from jax.experimental.pallas import tpu_sc as plsc

_B = 262144
_D = 256
_K = 5
_NC = 2
_NS = 16
_NW = _NC * _NS
_RW = _B // _NW
_C = 64
_G = _C // 16
_NCH = _RW // _C
_L = 16
_U = 4


def _round_bf16(v):
    u = plsc.bitcast(v, jnp.int32)
    r = (u + jnp.int32(0x8000)) & jnp.int32(-65536)
    return plsc.bitcast(r, jnp.float32)


def _sc_body(x_hbm, w_hbm, out_hbm, xbuf, wbuf, wrbuf, obuf, sem0, sem1):
    wid = lax.axis_index("c") * _NS + lax.axis_index("s")
    wbase = wid * _RW
    sems = (sem0, sem1)

    pltpu.sync_copy(w_hbm, wbuf)

    lanes = lax.iota(jnp.int32, _L)
    rowidx = [lanes + g * _L for g in range(_G)]
    ksplat = [jnp.broadcast_to(jnp.int32(k), (_L,)) for k in range(_K)]

    def wbody(d, w2accs):
        dsp = jnp.broadcast_to(d, (_L,))
        out = []
        for k in range(_K):
            wv = plsc.load_gather(wbuf, [ksplat[k], dsp])
            out.append(w2accs[k] + wv * wv)
        return tuple(out)

    w2s = lax.fori_loop(
        0, _D, wbody,
        tuple(jnp.zeros((_L,), jnp.float32) for _ in range(_K)))

    for k in range(_K):
        for v in range(_D // _L):
            wrbuf[k, pl.ds(v * _L, _L)] = _round_bf16(wbuf[k, pl.ds(v * _L, _L)])

    def fire(ci, slot):
        pltpu.make_async_copy(
            x_hbm.at[pl.ds(wbase + ci * _C, _C)],
            xbuf.at[slot],
            sems[slot],
        ).start()

    fire(0, 0)
    fire(1, 1)

    def process(ci, slot):
        pltpu.make_async_copy(
            x_hbm.at[pl.ds(wbase + ci * _C, _C)],
            xbuf.at[slot],
            sems[slot],
        ).wait()
        xr = xbuf.at[slot]

        init = tuple(
            tuple(jnp.zeros((_L,), jnp.float32) for _ in range(_K))
            for _ in range(_G)
        )

        @plsc.parallel_loop(0, _D, unroll=_U, carry=init)
        def accs(d, accs):
            dimidx = (jnp.broadcast_to(d, (_L,)) + lanes) & jnp.int32(_D - 1)
            wvals = [plsc.load_gather(wrbuf, [ksplat[k], dimidx])
                     for k in range(_K)]
            out = []
            for g in range(_G):
                xg = _round_bf16(plsc.load_gather(xr, [rowidx[g], dimidx]))
                out.append(tuple(accs[g][k] + xg * wvals[k]
                                 for k in range(_K)))
            return tuple(out)

        for g in range(_G):
            best = w2s[0] - 2.0 * accs[g][0]
            bi = jnp.zeros((_L,), jnp.int32)
            for k in range(1, _K):
                sk = w2s[k] - 2.0 * accs[g][k]
                m = sk < best
                best = jnp.where(m, sk, best)
                bi = jnp.where(m, jnp.int32(k), bi)
            obuf[pl.ds(ci * _C + g * _L, _L)] = bi

        @pl.when(ci + 2 < _NCH)
        def _():
            fire(ci + 2, slot)

    def outer(i, carry):
        for b in range(2):
            process(2 * i + b, b)
        return carry

    lax.fori_loop(0, _NCH // 2, outer, 0)

    pltpu.sync_copy(obuf, out_hbm.at[pl.ds(wbase, _RW)])


@jax.jit
def kernel(x, weights):
    xf = x
    wf = weights
    f = pl.kernel(
        _sc_body,
        out_type=jax.ShapeDtypeStruct((_B,), jnp.int32),
        mesh=plsc.VectorSubcoreMesh(core_axis_name="c", subcore_axis_name="s"),
        compiler_params=pltpu.CompilerParams(
            needs_layout_passes=False,
            use_tc_tiling_on_sc=False,
        ),
        scratch_types=[
            pltpu.VMEM((2, _C, _D), jnp.float32),
            pltpu.VMEM((_K, _D), jnp.float32),
            pltpu.VMEM((_K, _D), jnp.float32),
            pltpu.VMEM((_RW,), jnp.int32),
            pltpu.SemaphoreType.DMA,
            pltpu.SemaphoreType.DMA,
        ],
    )
    return f(xf, wf)


# speedup vs baseline: 8.3753x; 2.4002x over previous
import functools

import jax
import jax.numpy as jnp
from jax import lax
from jax.experimental import pallas as pl
from jax.experimental.pallas import tpu as pltpu
from jax.experimental.pallas import tpu_sc as plsc

_B = 262144
_D = 256
_K = 5

_BSC = 24576
_BLK = 8192
assert _BSC % _BLK == 0

_NC = 2
_NS = 16
_NW = _NC * _NS
_RW = _BSC // _NW
_C = 64
_G = _C // 16
_NCH = _RW // _C
_L = 16
_U = 4
assert _NCH % 2 == 0


def _round_bf16(v):
    u = plsc.bitcast(v, jnp.int32)
    r = (u + jnp.int32(0x8000)) & jnp.int32(-65536)
    return plsc.bitcast(r, jnp.float32)


def _sc_body(x_hbm, w_hbm, out_hbm, xbuf, wbuf, wrbuf, obuf, sem0, sem1):
    wid = lax.axis_index("c") * _NS + lax.axis_index("s")
    wbase = wid * _RW
    sems = (sem0, sem1)

    pltpu.sync_copy(w_hbm, wbuf)

    lanes = lax.iota(jnp.int32, _L)
    rowidx = [lanes + g * _L for g in range(_G)]
    ksplat = [jnp.broadcast_to(jnp.int32(k), (_L,)) for k in range(_K)]

    def wbody(d, w2accs):
        dsp = jnp.broadcast_to(d, (_L,))
        out = []
        for k in range(_K):
            wv = plsc.load_gather(wbuf, [ksplat[k], dsp])
            out.append(w2accs[k] + wv * wv)
        return tuple(out)

    w2s = lax.fori_loop(
        0, _D, wbody,
        tuple(jnp.zeros((_L,), jnp.float32) for _ in range(_K)))

    for k in range(_K):
        for v in range(_D // _L):
            wrbuf[k, pl.ds(v * _L, _L)] = _round_bf16(wbuf[k, pl.ds(v * _L, _L)])

    def fire(ci, slot):
        pltpu.make_async_copy(
            x_hbm.at[pl.ds(wbase + ci * _C, _C)],
            xbuf.at[slot],
            sems[slot],
        ).start()

    fire(0, 0)
    fire(1, 1)

    def process(ci, slot):
        pltpu.make_async_copy(
            x_hbm.at[pl.ds(wbase + ci * _C, _C)],
            xbuf.at[slot],
            sems[slot],
        ).wait()
        xr = xbuf.at[slot]

        init = tuple(
            tuple(jnp.zeros((_L,), jnp.float32) for _ in range(_K))
            for _ in range(_G)
        )

        @plsc.parallel_loop(0, _D, unroll=_U, carry=init)
        def accs(d, accs):
            dimidx = (jnp.broadcast_to(d, (_L,)) + lanes) & jnp.int32(_D - 1)
            wvals = [plsc.load_gather(wrbuf, [ksplat[k], dimidx])
                     for k in range(_K)]
            out = []
            for g in range(_G):
                xg = _round_bf16(plsc.load_gather(xr, [rowidx[g], dimidx]))
                out.append(tuple(accs[g][k] + xg * wvals[k]
                                 for k in range(_K)))
            return tuple(out)

        for g in range(_G):
            best = w2s[0] - 2.0 * accs[g][0]
            bi = jnp.zeros((_L,), jnp.int32)
            for k in range(1, _K):
                sk = w2s[k] - 2.0 * accs[g][k]
                m = sk < best
                best = jnp.where(m, sk, best)
                bi = jnp.where(m, jnp.int32(k), bi)
            obuf[pl.ds(ci * _C + g * _L, _L)] = bi

        @pl.when(ci + 2 < _NCH)
        def _():
            fire(ci + 2, slot)

    def outer(i, carry):
        for b in range(2):
            process(2 * i + b, b)
        return carry

    lax.fori_loop(0, _NCH // 2, outer, 0)

    pltpu.sync_copy(obuf, out_hbm.at[pl.ds(wbase, _RW)])


def _sc_call(x, weights):
    f = pl.kernel(
        _sc_body,
        out_type=jax.ShapeDtypeStruct((_BSC,), jnp.int32),
        mesh=plsc.VectorSubcoreMesh(core_axis_name="c", subcore_axis_name="s"),
        compiler_params=pltpu.CompilerParams(
            needs_layout_passes=False,
            use_tc_tiling_on_sc=False,
        ),
        scratch_types=[
            pltpu.VMEM((2, _C, _D), jnp.float32),
            pltpu.VMEM((_K, _D), jnp.float32),
            pltpu.VMEM((_K, _D), jnp.float32),
            pltpu.VMEM((_RW,), jnp.int32),
            pltpu.SemaphoreType.DMA,
            pltpu.SemaphoreType.DMA,
        ],
    )
    return f(x, weights)


def _tc_body(x_ref, w_ref, out_ref):
    xb = x_ref[...]
    wb = w_ref[...]
    dots = lax.dot_general(wb, xb, (((1,), (1,)), ((), ())),
                           preferred_element_type=jnp.float32)
    w2 = jnp.sum(wb * wb, axis=1, keepdims=True)
    s = w2 - 2.0 * dots
    k = s.shape[0]
    min_s = jnp.min(s, axis=0, keepdims=True)
    iota_k = lax.broadcasted_iota(jnp.int32, s.shape, 0)
    idx = jnp.min(jnp.where(s == min_s, iota_k, k), axis=0)
    out_ref[...] = idx.astype(jnp.int32)


def _tc_call(x, weights):
    b, d = x.shape
    off = _BSC // _BLK
    grid = ((b - _BSC) // _BLK,)
    return pl.pallas_call(
        _tc_body,
        grid=grid,
        in_specs=[
            pl.BlockSpec((_BLK, d), lambda i: (i + off, 0)),
            pl.BlockSpec(weights.shape, lambda i: (0, 0)),
        ],
        out_specs=pl.BlockSpec((_BLK,), lambda i: (i,)),
        out_shape=jax.ShapeDtypeStruct((b - _BSC,), jnp.int32),
        compiler_params=pltpu.CompilerParams(
            dimension_semantics=("arbitrary",),
        ),
    )(x, weights)


@jax.jit
def kernel(x, weights):
    out_sc = _sc_call(x, weights)
    out_tc = _tc_call(x, weights)
    return jnp.concatenate([out_sc, out_tc])


# speedup vs baseline: 25.2600x; 3.0160x over previous
import functools

import jax
import jax.numpy as jnp
from jax import lax
from jax.experimental import pallas as pl
from jax.experimental.pallas import tpu as pltpu
from jax.experimental.pallas import tpu_sc as plsc

_B = 262144
_D = 256
_K = 5

_BSC = 24576
_BLK = 8192
assert _BSC % _BLK == 0

_NC = 2
_NS = 16
_NW = _NC * _NS
_RW = _BSC // _NW
_C = 64
_G = _C // 16
_NCH = _RW // _C
_L = 16
_U = 4
assert _NCH % 2 == 0


def _round_bf16(v):
    u = plsc.bitcast(v, jnp.int32)
    r = (u + jnp.int32(0x8000)) & jnp.int32(-65536)
    return plsc.bitcast(r, jnp.float32)


def _sc_body(x_hbm, w_hbm, out_hbm, xbuf, wbuf, wrbuf, obuf, sem0, sem1):
    wid = lax.axis_index("c") * _NS + lax.axis_index("s")
    wbase = wid * _RW
    sems = (sem0, sem1)

    pltpu.sync_copy(w_hbm, wbuf)

    lanes = lax.iota(jnp.int32, _L)
    rowidx = [lanes + g * _L for g in range(_G)]
    ksplat = [jnp.broadcast_to(jnp.int32(k), (_L,)) for k in range(_K)]

    def wbody(d, w2accs):
        dsp = jnp.broadcast_to(d, (_L,))
        out = []
        for k in range(_K):
            wv = plsc.load_gather(wbuf, [ksplat[k], dsp])
            out.append(w2accs[k] + wv * wv)
        return tuple(out)

    w2s = lax.fori_loop(
        0, _D, wbody,
        tuple(jnp.zeros((_L,), jnp.float32) for _ in range(_K)))

    for k in range(_K):
        for v in range(_D // _L):
            wrbuf[k, pl.ds(v * _L, _L)] = _round_bf16(wbuf[k, pl.ds(v * _L, _L)])

    def fire(ci, slot):
        pltpu.make_async_copy(
            x_hbm.at[pl.ds(wbase + ci * _C, _C)],
            xbuf.at[slot],
            sems[slot],
        ).start()

    fire(0, 0)
    fire(1, 1)

    def process(ci, slot):
        pltpu.make_async_copy(
            x_hbm.at[pl.ds(wbase + ci * _C, _C)],
            xbuf.at[slot],
            sems[slot],
        ).wait()
        xr = xbuf.at[slot]

        init = tuple(
            tuple(jnp.zeros((_L,), jnp.float32) for _ in range(_K))
            for _ in range(_G)
        )

        @plsc.parallel_loop(0, _D, unroll=_U, carry=init)
        def accs(d, accs):
            dimidx = (jnp.broadcast_to(d, (_L,)) + lanes) & jnp.int32(_D - 1)
            wvals = [plsc.load_gather(wrbuf, [ksplat[k], dimidx])
                     for k in range(_K)]
            out = []
            for g in range(_G):
                xg = _round_bf16(plsc.load_gather(xr, [rowidx[g], dimidx]))
                out.append(tuple(accs[g][k] + xg * wvals[k]
                                 for k in range(_K)))
            return tuple(out)

        for g in range(_G):
            best = w2s[0] - 2.0 * accs[g][0]
            bi = jnp.zeros((_L,), jnp.int32)
            for k in range(1, _K):
                sk = w2s[k] - 2.0 * accs[g][k]
                m = sk < best
                best = jnp.where(m, sk, best)
                bi = jnp.where(m, jnp.int32(k), bi)
            obuf[pl.ds(ci * _C + g * _L, _L)] = bi

        @pl.when(ci + 2 < _NCH)
        def _():
            fire(ci + 2, slot)

    def outer(i, carry):
        for b in range(2):
            process(2 * i + b, b)
        return carry

    lax.fori_loop(0, _NCH // 2, outer, 0)

    pltpu.sync_copy(obuf, out_hbm.at[pl.ds(wbase, _RW)])


def _sc_call(x, weights):
    f = pl.kernel(
        _sc_body,
        out_type=jax.ShapeDtypeStruct((_BSC,), jnp.int32),
        mesh=plsc.VectorSubcoreMesh(core_axis_name="c", subcore_axis_name="s"),
        compiler_params=pltpu.CompilerParams(
            needs_layout_passes=False,
            use_tc_tiling_on_sc=True,
        ),
        scratch_types=[
            pltpu.VMEM((2, _C, _D), jnp.float32),
            pltpu.VMEM((_K, _D), jnp.float32),
            pltpu.VMEM((_K, _D), jnp.float32),
            pltpu.VMEM((_RW,), jnp.int32),
            pltpu.SemaphoreType.DMA,
            pltpu.SemaphoreType.DMA,
        ],
    )
    return f(x, weights)


def _tc_body(x_ref, w_ref, out_ref):
    xb = x_ref[...]
    wb = w_ref[...]
    dots = lax.dot_general(wb, xb, (((1,), (1,)), ((), ())),
                           preferred_element_type=jnp.float32)
    w2 = jnp.sum(wb * wb, axis=1, keepdims=True)
    s = w2 - 2.0 * dots
    k = s.shape[0]
    min_s = jnp.min(s, axis=0, keepdims=True)
    iota_k = lax.broadcasted_iota(jnp.int32, s.shape, 0)
    idx = jnp.min(jnp.where(s == min_s, iota_k, k), axis=0)
    out_ref[...] = idx.astype(jnp.int32)


def _tc_call(x, weights):
    b, d = x.shape
    off = _BSC // _BLK
    grid = ((b - _BSC) // _BLK,)
    return pl.pallas_call(
        _tc_body,
        grid=grid,
        in_specs=[
            pl.BlockSpec((_BLK, d), lambda i: (i + off, 0)),
            pl.BlockSpec(weights.shape, lambda i: (0, 0)),
        ],
        out_specs=pl.BlockSpec((_BLK,), lambda i: (i,)),
        out_shape=jax.ShapeDtypeStruct((b - _BSC,), jnp.int32),
        compiler_params=pltpu.CompilerParams(
            dimension_semantics=("arbitrary",),
        ),
    )(x, weights)


@jax.jit
def kernel(x, weights):
    out_sc = _sc_call(x, weights)
    out_tc = _tc_call(x, weights)
    return jnp.concatenate([out_sc, out_tc])
